# Initial kernel scaffold; baseline (speedup 1.0000x reference)
import functools

import jax
import jax.numpy as jnp
from jax import lax
from jax.experimental import pallas as pl
from jax.experimental.pallas import tpu as pltpu
from jax.experimental.pallas import tpu_sc as plsc

N = 10000
D = 128
NC = 2
NS = 16
NW = NC * NS
ROWS_PER_TILE = 626
N_PAD = NS * ROWS_PER_TILE
HALF = ROWS_PER_TILE // 2
E = 320000
K = 128
CHUNKS = 79
E_PAD = NW * CHUNKS * K
CW = 16


def _sc_aggregate_body(with_counts, x_hbm, src_hbm, dst_hbm, agg_hbm, cnt_hbm,
                       src_v, dst_v, rows_v, zbuf_v, cbuf_v, ones_v,
                       acc_sh, cnt_sh, sem):
    c = lax.axis_index("c")
    s = lax.axis_index("s")
    w = s * NC + c

    def _zrow(i, _):
        for k in range(8):
            zbuf_v[i, pl.ds(k * 16, 16)] = jnp.zeros((16,), jnp.float32)
        return 0
    lax.fori_loop(0, HALF, _zrow, 0)

    base = s * ROWS_PER_TILE
    pltpu.sync_copy(zbuf_v, acc_sh.at[pl.ds(base, HALF)])
    pltpu.sync_copy(zbuf_v, acc_sh.at[pl.ds(base + HALF, HALF)])

    if with_counts:
        def _zcrow(i, _):
            cbuf_v[i, pl.ds(0, 16)] = jnp.zeros((16,), jnp.float32)
            return 0
        lax.fori_loop(0, ROWS_PER_TILE, _zcrow, 0)
        pltpu.sync_copy(cbuf_v, cnt_sh.at[pl.ds(base, ROWS_PER_TILE)])
        for k in range(K // 16):
            ones_v[0, pl.ds(k * 16, 16)] = jnp.ones((16,), jnp.float32)

    plsc.subcore_barrier()

    pltpu.sync_copy(src_hbm.at[w], src_v)
    pltpu.sync_copy(dst_hbm.at[w], dst_v)

    def _chunk(j, _):
        pltpu.async_copy(x_hbm.at[src_v.at[j]], rows_v, sem).wait()
        pltpu.sync_copy(rows_v, acc_sh.at[dst_v.at[j]], add=True)
        if with_counts:
            pltpu.sync_copy(ones_v, cnt_sh.at[dst_v.at[j]], add=True)
        return 0
    lax.fori_loop(0, CHUNKS, _chunk, 0)

    plsc.subcore_barrier()

    pltpu.sync_copy(acc_sh.at[pl.ds(base, HALF)], zbuf_v)
    pltpu.sync_copy(zbuf_v, agg_hbm.at[c].at[pl.ds(base, HALF)])
    pltpu.sync_copy(acc_sh.at[pl.ds(base + HALF, HALF)], zbuf_v)
    pltpu.sync_copy(zbuf_v, agg_hbm.at[c].at[pl.ds(base + HALF, HALF)])
    if with_counts:
        pltpu.sync_copy(cnt_sh.at[pl.ds(base, ROWS_PER_TILE)], cbuf_v)
        pltpu.sync_copy(cbuf_v, cnt_hbm.at[c].at[pl.ds(base, ROWS_PER_TILE)])


def _make_sc_aggregate(with_counts):
    mesh = plsc.VectorSubcoreMesh(core_axis_name="c", subcore_axis_name="s")
    out_type = (
        jax.ShapeDtypeStruct((NC, N_PAD, D), jnp.float32),
        jax.ShapeDtypeStruct((NC, N_PAD, CW), jnp.float32),
    )
    scratch = [
        pltpu.VMEM((CHUNKS, K), jnp.int32),
        pltpu.VMEM((CHUNKS, K), jnp.int32),
        pltpu.VMEM((K, D), jnp.float32),
        pltpu.VMEM((HALF, D), jnp.float32),
        pltpu.VMEM((ROWS_PER_TILE, CW), jnp.float32),
        pltpu.VMEM((1, K), jnp.float32),
        pltpu.VMEM_SHARED((N_PAD, D), jnp.float32),
        pltpu.VMEM_SHARED((N_PAD, CW), jnp.float32),
        pltpu.SemaphoreType.DMA,
    ]
    return pl.kernel(
        functools.partial(_sc_aggregate_body, with_counts),
        out_type=out_type, mesh=mesh, scratch_types=scratch,
        name=f"sc_sage_aggregate_cnt{int(with_counts)}",
    )


_sc_agg_cnt = _make_sc_aggregate(True)
_sc_agg_nocnt = _make_sc_aggregate(False)

BR = 1000


def _tc_layer_body(act, agg_ref, cnt_ref, x_ref, wl_ref, bl_ref, wr_ref,
                   out_ref, sig_ref):
    agg = agg_ref[0] + agg_ref[1]
    cnt = cnt_ref[0, :, 0:1] + cnt_ref[1, :, 0:1]
    mean = agg * (1.0 / jnp.clip(cnt, 1.0, None))
    out = (jnp.dot(mean, wl_ref[...], preferred_element_type=jnp.float32)
           + bl_ref[...]
           + jnp.dot(x_ref[...], wr_ref[...], preferred_element_type=jnp.float32))
    if act == "relu":
        out_ref[...] = jnp.maximum(out, 0.0)
        sig_ref[...] = jnp.zeros_like(out)
    else:
        out_ref[...] = out
        sig_ref[...] = jax.nn.sigmoid(out)


def _make_tc_layer(act):
    grid = (N // BR,)
    in_specs = [
        pl.BlockSpec((NC, BR, D), lambda i: (0, i, 0)),
        pl.BlockSpec((NC, BR, CW), lambda i: (0, i, 0)),
        pl.BlockSpec((BR, D), lambda i: (i, 0)),
        pl.BlockSpec((D, D), lambda i: (0, 0)),
        pl.BlockSpec((1, D), lambda i: (0, 0)),
        pl.BlockSpec((D, D), lambda i: (0, 0)),
    ]
    out_specs = (pl.BlockSpec((BR, D), lambda i: (i, 0)),
                 pl.BlockSpec((BR, D), lambda i: (i, 0)))
    out_shape = (jax.ShapeDtypeStruct((N, D), jnp.float32),
                 jax.ShapeDtypeStruct((N, D), jnp.float32))
    return pl.pallas_call(
        functools.partial(_tc_layer_body, act),
        grid=grid, in_specs=in_specs, out_specs=out_specs,
        out_shape=out_shape,
    )


_tc_layer_relu = _make_tc_layer("relu")
_tc_layer_sig = _make_tc_layer("sig")


def kernel(x, edge_index, Wl0, bl0, Wr0, Wl1, bl1, Wr1):
    src = edge_index[0]
    dst = edge_index[1]
    pad = E_PAD - E
    src_r = jnp.concatenate([src, jnp.zeros((pad,), jnp.int32)]).reshape(NW, CHUNKS, K)
    dst_r = jnp.concatenate([dst, jnp.full((pad,), N, jnp.int32)]).reshape(NW, CHUNKS, K)

    agg0, cnt = _sc_agg_cnt(x, src_r, dst_r)
    h, _ = _tc_layer_relu(agg0, cnt, x, Wl0, bl0.reshape(1, D), Wr0)
    agg1, _ = _sc_agg_nocnt(h, src_r, dst_r)
    out, sig = _tc_layer_sig(agg1, cnt, h, Wl1, bl1.reshape(1, D), Wr1)
    return (out, sig)


# speedup vs baseline: 5.8900x; 5.8900x over previous
import functools

import jax
import jax.numpy as jnp
from jax import lax
from jax.experimental import pallas as pl
from jax.experimental.pallas import tpu as pltpu
from jax.experimental.pallas import tpu_sc as plsc

N = 10000
D = 128
DH = 64
NC = 2
NS = 16
ROWS_PER_TILE = 628
N_PAD = NS * ROWS_PER_TILE
HALF = ROWS_PER_TILE // 2
E = 320000
K = 128
CHUNKS = 157
E_PAD = NS * CHUNKS * K
CW = 16


def _sc_aggregate_body(with_counts, xlo_hbm, xhi_hbm, edges_hbm,
                       agglo_hbm, agghi_hbm, *refs):
    if with_counts:
        (cnt_hbm, src_v, dst_v, rows_v, zbuf_v, cbuf_v, ones_v,
         acc_sh, cnt_sh, sem) = refs
    else:
        (src_v, dst_v, rows_v, zbuf_v, acc_sh, sem) = refs
    c = lax.axis_index("c")
    s = lax.axis_index("s")

    def _zrow(i, _):
        for k in range(DH // 16):
            zbuf_v[i, pl.ds(k * 16, 16)] = jnp.zeros((16,), jnp.float32)
        return 0
    lax.fori_loop(0, HALF, _zrow, 0)

    base = s * ROWS_PER_TILE
    pltpu.sync_copy(zbuf_v, acc_sh.at[pl.ds(base, HALF)])
    pltpu.sync_copy(zbuf_v, acc_sh.at[pl.ds(base + HALF, HALF)])

    if with_counts:
        def _zcrow(i, _):
            cbuf_v[i, pl.ds(0, 16)] = jnp.zeros((16,), jnp.float32)
            return 0
        lax.fori_loop(0, ROWS_PER_TILE, _zcrow, 0)

        def _orow(i, _):
            ones_v[i, pl.ds(0, 16)] = jnp.ones((16,), jnp.float32)
            return 0
        lax.fori_loop(0, K, _orow, 0)

        @pl.when(c == 0)
        def _():
            pltpu.sync_copy(cbuf_v, cnt_sh.at[pl.ds(base, ROWS_PER_TILE)])

    plsc.subcore_barrier()

    pltpu.sync_copy(edges_hbm.at[s], src_v)

    def _unpack(i, _):
        for k in range(K // 16):
            v = src_v[i, pl.ds(k * 16, 16)]
            dst_v[i, pl.ds(k * 16, 16)] = lax.bitwise_and(v, 16383)
            src_v[i, pl.ds(k * 16, 16)] = lax.shift_right_logical(v, 14)
        return 0
    lax.fori_loop(0, CHUNKS, _unpack, 0)

    def _chunk_c0(j, _):
        pltpu.async_copy(xlo_hbm.at[src_v.at[j]], rows_v, sem).wait()
        pltpu.sync_copy(rows_v, acc_sh.at[dst_v.at[j]], add=True)
        if with_counts:
            pltpu.sync_copy(ones_v, cnt_sh.at[dst_v.at[j]], add=True)
        return 0

    def _chunk_c1(j, _):
        pltpu.async_copy(xhi_hbm.at[src_v.at[j]], rows_v, sem).wait()
        pltpu.sync_copy(rows_v, acc_sh.at[dst_v.at[j]], add=True)
        return 0

    @pl.when(c == 0)
    def _():
        lax.fori_loop(0, CHUNKS, _chunk_c0, 0)

    @pl.when(c == 1)
    def _():
        lax.fori_loop(0, CHUNKS, _chunk_c1, 0)

    plsc.subcore_barrier()

    @pl.when(c == 0)
    def _():
        pltpu.sync_copy(acc_sh.at[pl.ds(base, HALF)], zbuf_v)
        pltpu.sync_copy(zbuf_v, agglo_hbm.at[pl.ds(base, HALF)])
        pltpu.sync_copy(acc_sh.at[pl.ds(base + HALF, HALF)], zbuf_v)
        pltpu.sync_copy(zbuf_v, agglo_hbm.at[pl.ds(base + HALF, HALF)])
        if with_counts:
            pltpu.sync_copy(cnt_sh.at[pl.ds(base, ROWS_PER_TILE)], cbuf_v)
            pltpu.sync_copy(cbuf_v, cnt_hbm.at[pl.ds(base, ROWS_PER_TILE)])

    @pl.when(c == 1)
    def _():
        pltpu.sync_copy(acc_sh.at[pl.ds(base, HALF)], zbuf_v)
        pltpu.sync_copy(zbuf_v, agghi_hbm.at[pl.ds(base, HALF)])
        pltpu.sync_copy(acc_sh.at[pl.ds(base + HALF, HALF)], zbuf_v)
        pltpu.sync_copy(zbuf_v, agghi_hbm.at[pl.ds(base + HALF, HALF)])


def _make_sc_aggregate(with_counts):
    mesh = plsc.VectorSubcoreMesh(core_axis_name="c", subcore_axis_name="s")
    out_type = [
        jax.ShapeDtypeStruct((N_PAD, DH), jnp.float32),
        jax.ShapeDtypeStruct((N_PAD, DH), jnp.float32),
    ]
    scratch = [
        pltpu.VMEM((CHUNKS, K), jnp.int32),
        pltpu.VMEM((CHUNKS, K), jnp.int32),
        pltpu.VMEM((K, DH), jnp.float32),
        pltpu.VMEM((HALF, DH), jnp.float32),
    ]
    if with_counts:
        out_type.append(jax.ShapeDtypeStruct((N_PAD, CW), jnp.float32))
        scratch += [
            pltpu.VMEM((ROWS_PER_TILE, CW), jnp.float32),
            pltpu.VMEM((K, CW), jnp.float32),
        ]
    scratch.append(pltpu.VMEM_SHARED((N_PAD, DH), jnp.float32))
    if with_counts:
        scratch.append(pltpu.VMEM_SHARED((N_PAD, CW), jnp.float32))
    scratch.append(pltpu.SemaphoreType.DMA)
    out_type = tuple(out_type)
    return pl.kernel(
        functools.partial(_sc_aggregate_body, with_counts),
        out_type=out_type, mesh=mesh, scratch_types=scratch,
        compiler_params=pltpu.CompilerParams(use_tc_tiling_on_sc=False),
        name=f"sc_sage_aggregate_cnt{int(with_counts)}",
    )


_sc_agg_cnt = _make_sc_aggregate(True)

BR = 1000


def _tc_layer_body(act, agglo_ref, agghi_ref, cnt_ref, x_ref, wl_ref, bl_ref,
                   wr_ref, out_ref, *maybe_sig):
    agg = jnp.concatenate([agglo_ref[...], agghi_ref[...]], axis=1)
    cnt = cnt_ref[:, 0:1]
    mean = agg * (1.0 / jnp.clip(cnt, 1.0, None))
    out = (jnp.dot(mean, wl_ref[...], preferred_element_type=jnp.float32)
           + bl_ref[...]
           + jnp.dot(x_ref[...], wr_ref[...], preferred_element_type=jnp.float32))
    if act == "relu":
        out_ref[...] = jnp.maximum(out, 0.0)
    else:
        out_ref[...] = out
        maybe_sig[0][...] = jax.nn.sigmoid(out)


def _make_tc_layer(act):
    grid = (N // BR,)
    in_specs = [
        pl.BlockSpec((BR, DH), lambda i: (i, 0)),
        pl.BlockSpec((BR, DH), lambda i: (i, 0)),
        pl.BlockSpec((BR, CW), lambda i: (i, 0)),
        pl.BlockSpec((BR, D), lambda i: (i, 0)),
        pl.BlockSpec((D, D), lambda i: (0, 0)),
        pl.BlockSpec((1, D), lambda i: (0, 0)),
        pl.BlockSpec((D, D), lambda i: (0, 0)),
    ]
    nouts = 1 if act == "relu" else 2
    out_specs = tuple(pl.BlockSpec((BR, D), lambda i: (i, 0)) for _ in range(nouts))
    out_shape = tuple(jax.ShapeDtypeStruct((N, D), jnp.float32) for _ in range(nouts))
    return pl.pallas_call(
        functools.partial(_tc_layer_body, act),
        grid=grid, in_specs=in_specs, out_specs=out_specs,
        out_shape=out_shape,
    )


_tc_layer_relu = _make_tc_layer("relu")
_tc_layer_sig = _make_tc_layer("sig")


def kernel(x, edge_index, Wl0, bl0, Wr0, Wl1, bl1, Wr1):
    src = edge_index[0]
    dst = edge_index[1]
    pad = E_PAD - E
    packed = src * 16384 + dst
    edges = jnp.concatenate(
        [packed, jnp.full((pad,), N, jnp.int32)]).reshape(NS, CHUNKS, K)

    xlo, xhi = x[:, :DH], x[:, DH:]
    agg0lo, agg0hi, cnt = _sc_agg_cnt(xlo, xhi, edges)
    (h,) = _tc_layer_relu(agg0lo, agg0hi, cnt, x, Wl0, bl0.reshape(1, D), Wr0)
    agg1lo, agg1hi, _ = _sc_agg_cnt(h[:, :DH], h[:, DH:], edges)
    out, sig = _tc_layer_sig(agg1lo, agg1hi, cnt, h, Wl1, bl1.reshape(1, D), Wr1)
    return (out, sig)
